# baseline (speedup 1.0000x reference)
import jax
import jax.numpy as jnp
from jax import lax
from jax.experimental import pallas as pl
from jax.experimental.pallas import tpu as pltpu

_NT = (((1,), (1,)), ((), ()))
_TN = (((0,), (0,)), ((), ()))


def _gru_t(gi, gh, h):
    d = h.shape[0]
    i_r, i_z, i_n = gi[:d], gi[d:2 * d], gi[2 * d:]
    h_r, h_z, h_n = gh[:d], gh[d:2 * d], gh[2 * d:]
    r = jax.nn.sigmoid(i_r + h_r)
    z = jax.nn.sigmoid(i_z + h_z)
    ng = jnp.tanh(i_n + r * h_n)
    return (1.0 - z) * ng + z * h


def _fused_kernel(obs_ref, adj_ref, w1, b1, w2, b2, w3, b3, mw, mb,
                  wih_x, wih_m, whh, bih, bhh, cw, cb, rw, rb, aw, ab,
                  cls_ref, pred_ref, all_ref, feat_scr):
    f32 = jnp.float32
    j = pl.program_id(1)
    n_blk = feat_scr.shape[0]
    r_blk = feat_scr.shape[2]

    @pl.when(j == 0)
    def _message_passing():
        def leaky(v):
            return jnp.where(v >= 0, v, 0.01 * v)

        obs = obs_ref[...]
        adj = adj_ref[...]
        n = adj.shape[0]

        xt = leaky(lax.dot_general(w1[...], obs, _NT,
                                   preferred_element_type=f32) + b1[...])
        xt = leaky(jnp.dot(w2[...], xt, preferred_element_type=f32) + b2[...])
        xt = leaky(jnp.dot(w3[...], xt, preferred_element_type=f32) + b3[...])

        mb_v, bih_v, bhh_v = mb[...], bih[...], bhh[...]

        gi_x = jnp.dot(wih_x[...], xt, preferred_element_type=f32) + bih_v

        rowsum_t = lax.dot_general(jnp.ones((1, n), f32), adj, _NT,
                                   preferred_element_type=f32)
        msg_t = mb_v * rowsum_t
        gi = gi_x + jnp.dot(wih_m[...], msg_t, preferred_element_type=f32)
        gh = jnp.broadcast_to(bhh_v, gi.shape)
        h = _gru_t(gi, gh, jnp.zeros_like(msg_t))

        for _ in range(2):
            m_t = jnp.dot(mw[...], h, preferred_element_type=f32) + mb_v
            msg_t = lax.dot_general(m_t, adj, _NT, preferred_element_type=f32)
            gi = gi_x + jnp.dot(wih_m[...], msg_t, preferred_element_type=f32)
            gh = jnp.dot(whh[...], h, preferred_element_type=f32) + bhh_v
            h = _gru_t(gi, gh, h)

        neigh_t = lax.dot_general(h, adj, _NT, preferred_element_type=f32)
        glob_t = jnp.broadcast_to(jnp.mean(h, axis=1, keepdims=True), h.shape)
        feat = jnp.concatenate([h, neigh_t, glob_t], axis=0)
        for jj in range(n_blk):
            feat_scr[jj] = feat[:, jj * r_blk:(jj + 1) * r_blk]

    ft = feat_scr[j]
    cls_ref[...] = lax.dot_general(ft, cw[...], _TN,
                                   preferred_element_type=f32) + cb[...]
    pred_ref[...] = lax.dot_general(ft, rw[...], _TN,
                                    preferred_element_type=f32) + rb[...]
    all_ref[...] = lax.dot_general(ft, aw[...], _TN,
                                   preferred_element_type=f32) + ab[...]


def kernel(node_obs, node_adj, enc_W1, enc_b1, enc_W2, enc_b2, enc_W3, enc_b3,
           msg_W, msg_b, gru_Wih, gru_Whh, gru_bih, gru_bhh, cls_W, cls_b,
           reg_W, reg_b, all_W, all_b):
    B, N, F = node_obs.shape
    D = enc_W3.shape[0]
    C = cls_W.shape[0]
    R = 512
    J = N // R

    args = (
        node_obs, node_adj,
        enc_W1, enc_b1.reshape(-1, 1),
        enc_W2, enc_b2.reshape(-1, 1),
        enc_W3, enc_b3.reshape(-1, 1),
        msg_W, msg_b.reshape(-1, 1),
        gru_Wih[:, :D], gru_Wih[:, D:],
        gru_Whh,
        gru_bih.reshape(-1, 1), gru_bhh.reshape(-1, 1),
        cls_W.T, cls_b.reshape(1, -1),
        reg_W.T, reg_b.reshape(1, -1),
        all_W.T, all_b.reshape(1, -1),
    )
    in_specs = [
        pl.BlockSpec((None, N, F), lambda b, j: (b, 0, 0)),
        pl.BlockSpec((None, N, N), lambda b, j: (b, 0, 0)),
    ] + [
        pl.BlockSpec(a.shape, lambda b, j, nd=a.ndim: (0,) * nd)
        for a in args[2:]
    ]
    return pl.pallas_call(
        _fused_kernel,
        grid=(B, J),
        in_specs=in_specs,
        out_specs=(
            pl.BlockSpec((None, R, C), lambda b, j: (b, j, 0)),
            pl.BlockSpec((None, R, 1), lambda b, j: (b, j, 0)),
            pl.BlockSpec((None, R, N), lambda b, j: (b, j, 0)),
        ),
        out_shape=(
            jax.ShapeDtypeStruct((B, N, C), node_obs.dtype),
            jax.ShapeDtypeStruct((B, N, 1), node_obs.dtype),
            jax.ShapeDtypeStruct((B, N, N), node_obs.dtype),
        ),
        scratch_shapes=[pltpu.VMEM((J, 3 * D, R), jnp.float32)],
    )(*args)


# speedup vs baseline: 1.1211x; 1.1211x over previous
import jax
import jax.numpy as jnp
from jax import lax
from jax.experimental import pallas as pl

_NT = (((1,), (1,)), ((), ()))
_TN = (((0,), (0,)), ((), ()))


def _gru_t(gi, gh, h):
    d = h.shape[0]
    i_r, i_z, i_n = gi[:d], gi[d:2 * d], gi[2 * d:]
    h_r, h_z, h_n = gh[:d], gh[d:2 * d], gh[2 * d:]
    r = jax.nn.sigmoid(i_r + h_r)
    z = jax.nn.sigmoid(i_z + h_z)
    ng = jnp.tanh(i_n + r * h_n)
    return (1.0 - z) * ng + z * h


def _mp_kernel(obs_ref, adj0, adj1, adj2, adj3, w1, b1, w2, b2, w3, b3,
               mw, mb, wih_x, wih_m, whh, bih, bhh, feat_ref):
    f32 = jnp.float32
    adj_refs = (adj0, adj1, adj2, adj3)

    def leaky(v):
        return jnp.where(v >= 0, v, 0.01 * v)

    def adj_nt(lhs):
        return jnp.concatenate(
            [lax.dot_general(lhs, a[...], _NT, preferred_element_type=f32)
             for a in adj_refs], axis=1)

    obs = obs_ref[...]
    n = obs.shape[0]

    xt = leaky(lax.dot_general(w1[...], obs, _NT,
                               preferred_element_type=f32) + b1[...])
    xt = leaky(jnp.dot(w2[...], xt, preferred_element_type=f32) + b2[...])
    xt = leaky(jnp.dot(w3[...], xt, preferred_element_type=f32) + b3[...])

    mb_v, bih_v, bhh_v = mb[...], bih[...], bhh[...]

    gi_x = jnp.dot(wih_x[...], xt, preferred_element_type=f32) + bih_v

    rowsum_t = adj_nt(jnp.ones((1, n), f32))
    msg_t = mb_v * rowsum_t
    gi = gi_x + jnp.dot(wih_m[...], msg_t, preferred_element_type=f32)
    gh = jnp.broadcast_to(bhh_v, gi.shape)
    h = _gru_t(gi, gh, jnp.zeros_like(msg_t))

    for _ in range(2):
        m_t = jnp.dot(mw[...], h, preferred_element_type=f32) + mb_v
        msg_t = adj_nt(m_t)
        gi = gi_x + jnp.dot(wih_m[...], msg_t, preferred_element_type=f32)
        gh = jnp.dot(whh[...], h, preferred_element_type=f32) + bhh_v
        h = _gru_t(gi, gh, h)

    neigh_t = adj_nt(h)
    glob_t = jnp.broadcast_to(jnp.mean(h, axis=1, keepdims=True), h.shape)
    feat_ref[...] = jnp.concatenate([h, neigh_t, glob_t], axis=0)


def _readout_kernel(feat_ref, cw, cb, rw, rb, aw, ab,
                    cls_ref, pred_ref, all_ref):
    f32 = jnp.float32
    ft = feat_ref[...]
    cls_ref[...] = lax.dot_general(ft, cw[...], _TN,
                                   preferred_element_type=f32) + cb[...]
    pred_ref[...] = lax.dot_general(ft, rw[...], _TN,
                                    preferred_element_type=f32) + rb[...]
    all_ref[...] = lax.dot_general(ft, aw[...], _TN,
                                   preferred_element_type=f32) + ab[...]


def kernel(node_obs, node_adj, enc_W1, enc_b1, enc_W2, enc_b2, enc_W3, enc_b3,
           msg_W, msg_b, gru_Wih, gru_Whh, gru_bih, gru_bhh, cls_W, cls_b,
           reg_W, reg_b, all_W, all_b):
    B, N, F = node_obs.shape
    D = enc_W3.shape[0]
    C = cls_W.shape[0]

    mp_args = (
        node_obs, node_adj, node_adj, node_adj, node_adj,
        enc_W1, enc_b1.reshape(-1, 1),
        enc_W2, enc_b2.reshape(-1, 1),
        enc_W3, enc_b3.reshape(-1, 1),
        msg_W, msg_b.reshape(-1, 1),
        gru_Wih[:, :D], gru_Wih[:, D:],
        gru_Whh,
        gru_bih.reshape(-1, 1), gru_bhh.reshape(-1, 1),
    )
    mp_in_specs = [
        pl.BlockSpec((None, N, F), lambda b: (b, 0, 0)),
    ] + [
        pl.BlockSpec((None, N // 4, N), lambda b, q=q: (b, q, 0))
        for q in range(4)
    ] + [
        pl.BlockSpec(a.shape, lambda b, nd=a.ndim: (0,) * nd)
        for a in mp_args[5:]
    ]
    feat_t = pl.pallas_call(
        _mp_kernel,
        grid=(B,),
        in_specs=mp_in_specs,
        out_specs=pl.BlockSpec((None, 3 * D, N), lambda b: (b, 0, 0)),
        out_shape=jax.ShapeDtypeStruct((B, 3 * D, N), node_obs.dtype),
    )(*mp_args)

    R = 512
    ro_args = (
        feat_t,
        cls_W.T, cls_b.reshape(1, -1),
        reg_W.T, reg_b.reshape(1, -1),
        all_W.T, all_b.reshape(1, -1),
    )
    ro_in_specs = [
        pl.BlockSpec((None, 3 * D, R), lambda b, j: (b, 0, j)),
    ] + [
        pl.BlockSpec(a.shape, lambda b, j, nd=a.ndim: (0,) * nd)
        for a in ro_args[1:]
    ]
    cls, pred, pred_all = pl.pallas_call(
        _readout_kernel,
        grid=(B, N // R),
        in_specs=ro_in_specs,
        out_specs=(
            pl.BlockSpec((None, R, C), lambda b, j: (b, j, 0)),
            pl.BlockSpec((None, R, 1), lambda b, j: (b, j, 0)),
            pl.BlockSpec((None, R, N), lambda b, j: (b, j, 0)),
        ),
        out_shape=(
            jax.ShapeDtypeStruct((B, N, C), node_obs.dtype),
            jax.ShapeDtypeStruct((B, N, 1), node_obs.dtype),
            jax.ShapeDtypeStruct((B, N, N), node_obs.dtype),
        ),
    )(*ro_args)

    return (cls, pred, pred_all)


# speedup vs baseline: 1.1426x; 1.0192x over previous
import jax
import jax.numpy as jnp
from jax import lax
from jax.experimental import pallas as pl

_NT = (((1,), (1,)), ((), ()))
_TN = (((0,), (0,)), ((), ()))


def _gru_t(gi, gh, h):
    d = h.shape[0]
    i_r, i_z, i_n = gi[:d], gi[d:2 * d], gi[2 * d:]
    h_r, h_z, h_n = gh[:d], gh[d:2 * d], gh[2 * d:]
    r = jax.nn.sigmoid(i_r + h_r)
    z = jax.nn.sigmoid(i_z + h_z)
    ng = jnp.tanh(i_n + r * h_n)
    return (1.0 - z) * ng + z * h


def _mp_kernel(obs_ref, adj_ref, w1, b1, w2, b2, w3, b3, mw, mb, wih_x, wih_m,
               whh, bih, bhh, feat_ref):
    f32 = jnp.float32

    def leaky(v):
        return jnp.where(v >= 0, v, 0.01 * v)

    bf16 = jnp.bfloat16
    obs = obs_ref[...]
    adj = adj_ref[...].astype(bf16)
    n = adj.shape[0]

    xt = leaky(lax.dot_general(w1[...], obs, _NT,
                               preferred_element_type=f32) + b1[...])
    xt = leaky(jnp.dot(w2[...], xt, preferred_element_type=f32) + b2[...])
    xt = leaky(jnp.dot(w3[...], xt, preferred_element_type=f32) + b3[...])

    mb_v, bih_v, bhh_v = mb[...], bih[...], bhh[...]

    gi_x = jnp.dot(wih_x[...], xt, preferred_element_type=f32) + bih_v

    rowsum_t = lax.dot_general(jnp.ones((1, n), bf16), adj, _NT,
                               preferred_element_type=f32)
    msg_t = mb_v * rowsum_t
    gi = gi_x + jnp.dot(wih_m[...], msg_t, preferred_element_type=f32)
    gh = jnp.broadcast_to(bhh_v, gi.shape)
    h = _gru_t(gi, gh, jnp.zeros_like(msg_t))

    for _ in range(2):
        m_t = jnp.dot(mw[...], h, preferred_element_type=f32) + mb_v
        msg_t = lax.dot_general(m_t.astype(bf16), adj, _NT,
                                preferred_element_type=f32)
        gi = gi_x + jnp.dot(wih_m[...], msg_t, preferred_element_type=f32)
        gh = jnp.dot(whh[...], h, preferred_element_type=f32) + bhh_v
        h = _gru_t(gi, gh, h)

    neigh_t = lax.dot_general(h.astype(bf16), adj, _NT,
                              preferred_element_type=f32)
    glob_t = jnp.broadcast_to(jnp.mean(h, axis=1, keepdims=True), h.shape)
    feat_ref[...] = jnp.concatenate([h, neigh_t, glob_t], axis=0)


def _readout_kernel(feat_ref, cw, cb, rw, rb, aw, ab,
                    cls_ref, pred_ref, all_ref):
    f32 = jnp.float32
    ft = feat_ref[...]
    cls_ref[...] = lax.dot_general(ft, cw[...], _TN,
                                   preferred_element_type=f32) + cb[...]
    pred_ref[...] = lax.dot_general(ft, rw[...], _TN,
                                    preferred_element_type=f32) + rb[...]
    all_ref[...] = lax.dot_general(ft, aw[...], _TN,
                                   preferred_element_type=f32) + ab[...]


def kernel(node_obs, node_adj, enc_W1, enc_b1, enc_W2, enc_b2, enc_W3, enc_b3,
           msg_W, msg_b, gru_Wih, gru_Whh, gru_bih, gru_bhh, cls_W, cls_b,
           reg_W, reg_b, all_W, all_b):
    B, N, F = node_obs.shape
    D = enc_W3.shape[0]
    C = cls_W.shape[0]

    mp_args = (
        node_obs, node_adj,
        enc_W1, enc_b1.reshape(-1, 1),
        enc_W2, enc_b2.reshape(-1, 1),
        enc_W3, enc_b3.reshape(-1, 1),
        msg_W, msg_b.reshape(-1, 1),
        gru_Wih[:, :D], gru_Wih[:, D:],
        gru_Whh,
        gru_bih.reshape(-1, 1), gru_bhh.reshape(-1, 1),
    )
    mp_in_specs = [
        pl.BlockSpec((None, N, F), lambda b: (b, 0, 0)),
        pl.BlockSpec((None, N, N), lambda b: (b, 0, 0)),
    ] + [
        pl.BlockSpec(a.shape, lambda b, nd=a.ndim: (0,) * nd)
        for a in mp_args[2:]
    ]
    feat_t = pl.pallas_call(
        _mp_kernel,
        grid=(B,),
        in_specs=mp_in_specs,
        out_specs=pl.BlockSpec((None, 3 * D, N), lambda b: (b, 0, 0)),
        out_shape=jax.ShapeDtypeStruct((B, 3 * D, N), node_obs.dtype),
    )(*mp_args)

    R = 512
    ro_args = (
        feat_t,
        cls_W.T, cls_b.reshape(1, -1),
        reg_W.T, reg_b.reshape(1, -1),
        all_W.T, all_b.reshape(1, -1),
    )
    ro_in_specs = [
        pl.BlockSpec((None, 3 * D, R), lambda b, j: (b, 0, j)),
    ] + [
        pl.BlockSpec(a.shape, lambda b, j, nd=a.ndim: (0,) * nd)
        for a in ro_args[1:]
    ]
    cls, pred, pred_all = pl.pallas_call(
        _readout_kernel,
        grid=(B, N // R),
        in_specs=ro_in_specs,
        out_specs=(
            pl.BlockSpec((None, R, C), lambda b, j: (b, j, 0)),
            pl.BlockSpec((None, R, 1), lambda b, j: (b, j, 0)),
            pl.BlockSpec((None, R, N), lambda b, j: (b, j, 0)),
        ),
        out_shape=(
            jax.ShapeDtypeStruct((B, N, C), node_obs.dtype),
            jax.ShapeDtypeStruct((B, N, 1), node_obs.dtype),
            jax.ShapeDtypeStruct((B, N, N), node_obs.dtype),
        ),
    )(*ro_args)

    return (cls, pred, pred_all)


# speedup vs baseline: 1.1437x; 1.0009x over previous
import jax
import jax.numpy as jnp
from jax import lax
from jax.experimental import pallas as pl
from jax.experimental.pallas import tpu as pltpu

_NT = (((1,), (1,)), ((), ()))
_TN = (((0,), (0,)), ((), ()))


def _gru_t(gi, gh, h):
    d = h.shape[0]
    i_r, i_z, i_n = gi[:d], gi[d:2 * d], gi[2 * d:]
    h_r, h_z, h_n = gh[:d], gh[d:2 * d], gh[2 * d:]
    r = jax.nn.sigmoid(i_r + h_r)
    z = jax.nn.sigmoid(i_z + h_z)
    ng = jnp.tanh(i_n + r * h_n)
    return (1.0 - z) * ng + z * h


def _mp_kernel(obs_ref, adj_ref, w1, b1, w2, b2, w3, b3, mw, mb, wih_x, wih_m,
               whh, bih, bhh, feat_ref):
    f32 = jnp.float32

    def leaky(v):
        return jnp.where(v >= 0, v, 0.01 * v)

    bf16 = jnp.bfloat16
    obs = obs_ref[...]
    adj = adj_ref[...].astype(bf16)
    n = adj.shape[0]

    xt = leaky(lax.dot_general(w1[...], obs, _NT,
                               preferred_element_type=f32) + b1[...])
    xt = leaky(jnp.dot(w2[...], xt, preferred_element_type=f32) + b2[...])
    xt = leaky(jnp.dot(w3[...], xt, preferred_element_type=f32) + b3[...])

    mb_v, bih_v, bhh_v = mb[...], bih[...], bhh[...]

    gi_x = jnp.dot(wih_x[...], xt, preferred_element_type=f32) + bih_v

    rowsum_t = lax.dot_general(jnp.ones((1, n), bf16), adj, _NT,
                               preferred_element_type=f32)
    msg_t = mb_v * rowsum_t
    gi = gi_x + jnp.dot(wih_m[...], msg_t, preferred_element_type=f32)
    gh = jnp.broadcast_to(bhh_v, gi.shape)
    h = _gru_t(gi, gh, jnp.zeros_like(msg_t))

    for _ in range(2):
        m_t = jnp.dot(mw[...], h, preferred_element_type=f32) + mb_v
        msg_t = lax.dot_general(m_t.astype(bf16), adj, _NT,
                                preferred_element_type=f32)
        gi = gi_x + jnp.dot(wih_m[...], msg_t, preferred_element_type=f32)
        gh = jnp.dot(whh[...], h, preferred_element_type=f32) + bhh_v
        h = _gru_t(gi, gh, h)

    neigh_t = lax.dot_general(h.astype(bf16), adj, _NT,
                              preferred_element_type=f32)
    glob_t = jnp.broadcast_to(jnp.mean(h, axis=1, keepdims=True), h.shape)
    feat_ref[...] = jnp.concatenate([h, neigh_t, glob_t], axis=0)


def _readout_kernel(feat_ref, cw, cb, rw, rb, aw, ab,
                    cls_ref, pred_ref, all_ref):
    f32 = jnp.float32
    ft = feat_ref[...]
    cls_ref[...] = lax.dot_general(ft, cw[...], _TN,
                                   preferred_element_type=f32) + cb[...]
    pred_ref[...] = lax.dot_general(ft, rw[...], _TN,
                                    preferred_element_type=f32) + rb[...]
    all_ref[...] = lax.dot_general(ft, aw[...], _TN,
                                   preferred_element_type=f32) + ab[...]


def kernel(node_obs, node_adj, enc_W1, enc_b1, enc_W2, enc_b2, enc_W3, enc_b3,
           msg_W, msg_b, gru_Wih, gru_Whh, gru_bih, gru_bhh, cls_W, cls_b,
           reg_W, reg_b, all_W, all_b):
    B, N, F = node_obs.shape
    D = enc_W3.shape[0]
    C = cls_W.shape[0]

    mp_args = (
        node_obs, node_adj,
        enc_W1, enc_b1.reshape(-1, 1),
        enc_W2, enc_b2.reshape(-1, 1),
        enc_W3, enc_b3.reshape(-1, 1),
        msg_W, msg_b.reshape(-1, 1),
        gru_Wih[:, :D], gru_Wih[:, D:],
        gru_Whh,
        gru_bih.reshape(-1, 1), gru_bhh.reshape(-1, 1),
    )
    mp_in_specs = [
        pl.BlockSpec((None, N, F), lambda b: (b, 0, 0)),
        pl.BlockSpec((None, N, N), lambda b: (b, 0, 0)),
    ] + [
        pl.BlockSpec(a.shape, lambda b, nd=a.ndim: (0,) * nd)
        for a in mp_args[2:]
    ]
    feat_t = pl.pallas_call(
        _mp_kernel,
        grid=(B,),
        in_specs=mp_in_specs,
        out_specs=pl.BlockSpec((None, 3 * D, N), lambda b: (b, 0, 0)),
        out_shape=jax.ShapeDtypeStruct((B, 3 * D, N), node_obs.dtype),
        compiler_params=pltpu.CompilerParams(
            dimension_semantics=("parallel",)),
    )(*mp_args)

    R = 512
    ro_args = (
        feat_t,
        cls_W.T, cls_b.reshape(1, -1),
        reg_W.T, reg_b.reshape(1, -1),
        all_W.T, all_b.reshape(1, -1),
    )
    ro_in_specs = [
        pl.BlockSpec((None, 3 * D, R), lambda b, j: (b, 0, j)),
    ] + [
        pl.BlockSpec(a.shape, lambda b, j, nd=a.ndim: (0,) * nd)
        for a in ro_args[1:]
    ]
    cls, pred, pred_all = pl.pallas_call(
        _readout_kernel,
        grid=(B, N // R),
        in_specs=ro_in_specs,
        out_specs=(
            pl.BlockSpec((None, R, C), lambda b, j: (b, j, 0)),
            pl.BlockSpec((None, R, 1), lambda b, j: (b, j, 0)),
            pl.BlockSpec((None, R, N), lambda b, j: (b, j, 0)),
        ),
        out_shape=(
            jax.ShapeDtypeStruct((B, N, C), node_obs.dtype),
            jax.ShapeDtypeStruct((B, N, 1), node_obs.dtype),
            jax.ShapeDtypeStruct((B, N, N), node_obs.dtype),
        ),
        compiler_params=pltpu.CompilerParams(
            dimension_semantics=("parallel", "parallel")),
    )(*ro_args)

    return (cls, pred, pred_all)


# speedup vs baseline: 1.1746x; 1.0270x over previous
import jax
import jax.numpy as jnp
from jax import lax
from jax.experimental import pallas as pl

_NT = (((1,), (1,)), ((), ()))
_TN = (((0,), (0,)), ((), ()))


def _gru_t(gi, gh, h):
    d = h.shape[0]
    i_r, i_z, i_n = gi[:d], gi[d:2 * d], gi[2 * d:]
    h_r, h_z, h_n = gh[:d], gh[d:2 * d], gh[2 * d:]
    r = jax.nn.sigmoid(i_r + h_r)
    z = jax.nn.sigmoid(i_z + h_z)
    ng = jnp.tanh(i_n + r * h_n)
    return (1.0 - z) * ng + z * h


def _mp_kernel(obs_ref, adj_ref, w1, b1, w2, b2, w3, b3, mw, mb, wih_x, wih_m,
               whh, bih, bhh, feat_ref):
    f32 = jnp.float32

    def leaky(v):
        return jnp.where(v >= 0, v, 0.01 * v)

    obs = obs_ref[...]
    adj = adj_ref[...]
    n = adj.shape[0]

    xt = leaky(lax.dot_general(w1[...], obs, _NT,
                               preferred_element_type=f32) + b1[...])
    xt = leaky(jnp.dot(w2[...], xt, preferred_element_type=f32) + b2[...])
    xt = leaky(jnp.dot(w3[...], xt, preferred_element_type=f32) + b3[...])

    mb_v, bih_v, bhh_v = mb[...], bih[...], bhh[...]

    gi_x = jnp.dot(wih_x[...], xt, preferred_element_type=f32) + bih_v

    rowsum_t = lax.dot_general(jnp.ones((1, n), f32), adj, _NT,
                               preferred_element_type=f32)
    msg_t = mb_v * rowsum_t
    gi = gi_x + jnp.dot(wih_m[...], msg_t, preferred_element_type=f32)
    gh = jnp.broadcast_to(bhh_v, gi.shape)
    h = _gru_t(gi, gh, jnp.zeros_like(msg_t))

    for _ in range(2):
        m_t = jnp.dot(mw[...], h, preferred_element_type=f32) + mb_v
        msg_t = lax.dot_general(m_t, adj, _NT, preferred_element_type=f32)
        gi = gi_x + jnp.dot(wih_m[...], msg_t, preferred_element_type=f32)
        gh = jnp.dot(whh[...], h, preferred_element_type=f32) + bhh_v
        h = _gru_t(gi, gh, h)

    neigh_t = lax.dot_general(h, adj, _NT, preferred_element_type=f32)
    glob_t = jnp.broadcast_to(jnp.mean(h, axis=1, keepdims=True), h.shape)
    feat_ref[...] = jnp.concatenate([h, neigh_t, glob_t], axis=0)


def _readout_kernel(feat_ref, cw, cb, rw, rb, aw, ab,
                    cls_ref, pred_ref, all_ref):
    f32 = jnp.float32
    ft = feat_ref[...]
    cls_ref[...] = lax.dot_general(ft, cw[...], _TN,
                                   preferred_element_type=f32) + cb[...]
    pred_ref[...] = lax.dot_general(ft, rw[...], _TN,
                                    preferred_element_type=f32) + rb[...]
    all_ref[...] = lax.dot_general(ft, aw[...], _TN,
                                   preferred_element_type=f32) + ab[...]


def kernel(node_obs, node_adj, enc_W1, enc_b1, enc_W2, enc_b2, enc_W3, enc_b3,
           msg_W, msg_b, gru_Wih, gru_Whh, gru_bih, gru_bhh, cls_W, cls_b,
           reg_W, reg_b, all_W, all_b):
    B, N, F = node_obs.shape
    D = enc_W3.shape[0]
    C = cls_W.shape[0]

    mp_args = (
        node_obs, node_adj,
        enc_W1, enc_b1.reshape(-1, 1),
        enc_W2, enc_b2.reshape(-1, 1),
        enc_W3, enc_b3.reshape(-1, 1),
        msg_W, msg_b.reshape(-1, 1),
        gru_Wih[:, :D], gru_Wih[:, D:],
        gru_Whh,
        gru_bih.reshape(-1, 1), gru_bhh.reshape(-1, 1),
    )
    mp_in_specs = [
        pl.BlockSpec((None, N, F), lambda b: (b, 0, 0)),
        pl.BlockSpec((None, N, N), lambda b: (b, 0, 0)),
    ] + [
        pl.BlockSpec(a.shape, lambda b, nd=a.ndim: (0,) * nd)
        for a in mp_args[2:]
    ]
    feat_t = pl.pallas_call(
        _mp_kernel,
        grid=(B,),
        in_specs=mp_in_specs,
        out_specs=pl.BlockSpec((None, 3 * D, N), lambda b: (b, 0, 0)),
        out_shape=jax.ShapeDtypeStruct((B, 3 * D, N), node_obs.dtype),
    )(*mp_args)

    R = 1024
    ro_args = (
        feat_t,
        cls_W.T, cls_b.reshape(1, -1),
        reg_W.T, reg_b.reshape(1, -1),
        all_W.T, all_b.reshape(1, -1),
    )
    ro_in_specs = [
        pl.BlockSpec((None, 3 * D, R), lambda b, j: (b, 0, j)),
    ] + [
        pl.BlockSpec(a.shape, lambda b, j, nd=a.ndim: (0,) * nd)
        for a in ro_args[1:]
    ]
    cls, pred, pred_all = pl.pallas_call(
        _readout_kernel,
        grid=(B, N // R),
        in_specs=ro_in_specs,
        out_specs=(
            pl.BlockSpec((None, R, C), lambda b, j: (b, j, 0)),
            pl.BlockSpec((None, R, 1), lambda b, j: (b, j, 0)),
            pl.BlockSpec((None, R, N), lambda b, j: (b, j, 0)),
        ),
        out_shape=(
            jax.ShapeDtypeStruct((B, N, C), node_obs.dtype),
            jax.ShapeDtypeStruct((B, N, 1), node_obs.dtype),
            jax.ShapeDtypeStruct((B, N, N), node_obs.dtype),
        ),
    )(*ro_args)

    return (cls, pred, pred_all)
